# baseline (speedup 1.0000x reference)
import jax
import jax.numpy as jnp
from jax import lax
from jax.experimental import pallas as pl
from jax.experimental.pallas import tpu as pltpu
from jax.experimental.pallas import tpu_sc as plsc

_NC = 2
_NS = 16
_NW = _NC * _NS

_B = 4096
_L = 50
_E = 64
_TOT = _B * _L
_C = 128
_ROWS_PER_W = _TOT // _NW
_CH = _ROWS_PER_W // _C
_G = 5
_NG = _CH // _G


def _emb_body(src_idx, tgt_idx, src_tab, tgt_tab, outs,
              idxs, idxt, buf0, buf1, s0, s1, w0, w1):
    wid = lax.axis_index("s") * _NC + lax.axis_index("c")
    row0 = wid * _ROWS_PER_W

    pltpu.sync_copy(src_idx.at[pl.ds(wid * _CH, _CH)], idxs)
    pltpu.sync_copy(tgt_idx.at[pl.ds(wid * _CH, _CH)], idxt)

    def fire_group(tab, idxv, g, buf, sem):
        for k in range(_G):
            pltpu.async_copy(tab.at[idxv.at[g * _G + k]],
                             buf.at[pl.ds(k * _C, _C)], sem)

    def drain_group(tab, idxv, g, buf, sem):
        for k in range(_G):
            pltpu.make_async_copy(tab.at[idxv.at[g * _G + k]],
                                  buf.at[pl.ds(k * _C, _C)], sem).wait()

    def run_table(tab, out, idxv):
        def out_block(g):
            return out.at[pl.ds(row0 + g * _G * _C, _G * _C)]

        @pl.loop(0, _NG, step=2)
        def _pair(g):
            @pl.when(g >= 2)
            def _():
                pltpu.make_async_copy(buf0, out_block(g - 2), w0).wait()

            fire_group(tab, idxv, g, buf0, s0)

            @pl.when(g >= 1)
            def _():
                pltpu.make_async_copy(buf1, out_block(g - 1), w1).wait()

            fire_group(tab, idxv, g + 1, buf1, s1)
            drain_group(tab, idxv, g, buf0, s0)
            pltpu.async_copy(buf0, out_block(g), w0)
            drain_group(tab, idxv, g + 1, buf1, s1)
            pltpu.async_copy(buf1, out_block(g + 1), w1)

        pltpu.make_async_copy(buf0, out_block(_NG - 2), w0).wait()
        pltpu.make_async_copy(buf1, out_block(_NG - 1), w1).wait()

    run_table(src_tab, outs.at[0], idxs)
    run_table(tgt_tab, outs.at[1], idxt)


@jax.jit
def _emb(src_idx2d, tgt_idx2d, src_table, tgt_table):
    mesh = plsc.VectorSubcoreMesh(core_axis_name="c", subcore_axis_name="s")
    out_type = jax.ShapeDtypeStruct((2, _TOT, _E), jnp.float32)
    scratch = [
        pltpu.VMEM((_CH, _C), jnp.int32),
        pltpu.VMEM((_CH, _C), jnp.int32),
        pltpu.VMEM((_G * _C, _E), jnp.float32),
        pltpu.VMEM((_G * _C, _E), jnp.float32),
        pltpu.SemaphoreType.DMA,
        pltpu.SemaphoreType.DMA,
        pltpu.SemaphoreType.DMA,
        pltpu.SemaphoreType.DMA,
    ]
    fn = pl.kernel(_emb_body, out_type=out_type, mesh=mesh,
                   scratch_types=scratch,
                   compiler_params=pltpu.CompilerParams(
                       use_tc_tiling_on_sc=False))
    return fn(src_idx2d, tgt_idx2d, src_table, tgt_table)


def kernel(src_tokens, tgt_tokens, src_table, tgt_table):
    src_idx2d = src_tokens.astype(jnp.int32).reshape(_NW * _CH, _C)
    tgt_idx2d = tgt_tokens.astype(jnp.int32).reshape(_NW * _CH, _C)
    outs = _emb(src_idx2d, tgt_idx2d, src_table, tgt_table)
    return (outs[0].reshape(_B, _L, _E), outs[1].reshape(_B, _L, _E))
